# baseline (speedup 1.0000x reference)
import functools

import jax
import jax.numpy as jnp
from jax import lax
from jax.experimental import pallas as pl
from jax.experimental.pallas import tpu as pltpu
from jax.experimental.pallas import tpu_sc as plsc

NC = 2
NS = 16
LANES = 16

CHUNK = 80
EBLK = 2000
IBLK = 25


def _tc_matmul_split(x, w):
  m, k = x.shape
  _, n = w.shape
  hd = n // NC
  bm = 2000

  def body(x_ref, w_ref, o_ref):
    o_ref[...] = jnp.dot(x_ref[...], w_ref[...],
                         preferred_element_type=jnp.float32)

  return pl.pallas_call(
      body,
      grid=(NC, m // bm),
      in_specs=[
          pl.BlockSpec((bm, k), lambda c, i: (i, 0)),
          pl.BlockSpec((k, hd), lambda c, i: (0, c)),
      ],
      out_specs=pl.BlockSpec((bm, hd), lambda c, i: (c * (m // bm) + i, 0)),
      out_shape=jax.ShapeDtypeStruct((NC * m, hd), jnp.float32),
  )(x, w)


def _sc_aggregate(support2, src, dst, ew, b, n, d):
  hd = d // NC
  e = src.shape[0]
  ept = e // NS
  nblk = ept // EBLK
  nchunk = EBLK // CHUNK
  ninit = n // NS
  dcol = hd // LANES

  mesh = plsc.VectorSubcoreMesh(core_axis_name="c", subcore_axis_name="s")

  @functools.partial(
      pl.kernel,
      out_type=jax.ShapeDtypeStruct((n, d), jnp.float32),
      mesh=mesh,
      compiler_params=pltpu.CompilerParams(needs_layout_passes=False),
      scratch_types=dict(
          acc=pltpu.VMEM_SHARED((n, hd), jnp.float32),
          srcv=pltpu.VMEM((EBLK,), jnp.int32),
          dstv=pltpu.VMEM((EBLK,), jnp.int32),
          ewv=pltpu.VMEM((EBLK,), jnp.float32),
          idxv0=pltpu.VMEM((CHUNK,), jnp.int32),
          idxv1=pltpu.VMEM((CHUNK,), jnp.int32),
          idxv2=pltpu.VMEM((CHUNK,), jnp.int32),
          rows0=pltpu.VMEM((CHUNK, hd), jnp.float32),
          rows1=pltpu.VMEM((CHUNK, hd), jnp.float32),
          rows2=pltpu.VMEM((CHUNK, hd), jnp.float32),
          binit=pltpu.VMEM((IBLK, hd), jnp.float32),
          bvm=pltpu.VMEM((hd,), jnp.float32),
          gsem0=pltpu.SemaphoreType.DMA,
          gsem1=pltpu.SemaphoreType.DMA,
          gsem2=pltpu.SemaphoreType.DMA,
          ssem0=pltpu.SemaphoreType.DMA,
          ssem1=pltpu.SemaphoreType.DMA,
          ssem2=pltpu.SemaphoreType.DMA,
      ),
  )
  def agg(sup_hbm, src_hbm, dst_hbm, ew_hbm, b_hbm, out_hbm,
          acc, srcv, dstv, ewv, idxv0, idxv1, idxv2, rows0, rows1, rows2,
          binit, bvm, gsem0, gsem1, gsem2, ssem0, ssem1, ssem2):
    cid = lax.axis_index("c")
    sid = lax.axis_index("s")

    pltpu.sync_copy(b_hbm.at[pl.ds(cid * hd, hd)], bvm)

    def binit_body(i, _):
      for r in range(dcol):
        sl = pl.ds(r * LANES, LANES)
        binit[i, sl] = bvm[sl]
      return 0

    lax.fori_loop(0, IBLK, binit_body, 0)

    for j in range(ninit // IBLK):
      pltpu.sync_copy(binit, acc.at[pl.ds(sid * ninit + j * IBLK, IBLK)])

    plsc.subcore_barrier()

    row_off = cid * n

    bufs = ((idxv0, rows0, gsem0, ssem0),
            (idxv1, rows1, gsem1, ssem1),
            (idxv2, rows2, gsem2, ssem2))
    NBUF = len(bufs)

    def fill_idx(c, idxv):
      cb = c * CHUNK
      for t in range(CHUNK // LANES):
        sl = pl.ds(t * LANES, LANES)
        idxv[sl] = srcv[pl.ds(cb + t * LANES, LANES)] + row_off

    def block_body(bi, _):
      ebase = sid * ept + bi * EBLK
      pltpu.sync_copy(src_hbm.at[pl.ds(ebase, EBLK)], srcv)
      pltpu.sync_copy(dst_hbm.at[pl.ds(ebase, EBLK)], dstv)
      pltpu.sync_copy(ew_hbm.at[pl.ds(ebase, EBLK)], ewv)

      fill_idx(0, idxv0)
      gpend = {0: pltpu.async_copy(sup_hbm.at[idxv0], rows0, gsem0)}
      spend = {}
      for c in range(nchunk):
        cb = c * CHUNK
        idxv, rows, _, ssem = bufs[c % NBUF]
        gpend.pop(c).wait()
        if c + 1 < nchunk:
          nidxv, nrows, ngsem, _ = bufs[(c + 1) % NBUF]
          if c - 2 >= 0:
            spend.pop(c - 2).wait()
          fill_idx(c + 1, nidxv)
          gpend[c + 1] = pltpu.async_copy(sup_hbm.at[nidxv], nrows, ngsem)

        def scale_body(t, _, cb=cb, rows=rows):
          wv = ewv[pl.ds(cb + t * LANES, LANES)]
          for li in range(LANES):
            w = wv[li]
            rr = t * LANES + li
            for r in range(dcol):
              sl = pl.ds(r * LANES, LANES)
              rows[rr, sl] = rows[rr, sl] * w
          return 0

        lax.fori_loop(0, CHUNK // LANES, scale_body, 0)
        spend[c] = pltpu.async_copy(
            rows, acc.at[dstv.at[pl.ds(cb, CHUNK)]], ssem, add=True)
      for c in sorted(spend):
        spend.pop(c).wait()
      return 0

    lax.fori_loop(0, nblk, block_body, 0)

    plsc.subcore_barrier()

    wa = 624
    wlast = n - (NS - 1) * wa

    @pl.when(sid < NS - 1)
    def _():
      pltpu.sync_copy(
          acc.at[pl.ds(sid * wa, wa)],
          out_hbm.at[pl.ds(sid * wa, wa), pl.ds(cid * hd, hd)])

    @pl.when(sid == NS - 1)
    def _():
      pltpu.sync_copy(
          acc.at[pl.ds((NS - 1) * wa, wlast)],
          out_hbm.at[pl.ds((NS - 1) * wa, wlast), pl.ds(cid * hd, hd)])

  return agg(support2, src, dst, ew, b)


def kernel(x, edge_index, edge_weight, W, b):
  n, _ = x.shape
  d = W.shape[1]
  support2 = _tc_matmul_split(x, W)
  return _sc_aggregate(support2, edge_index[0], edge_index[1], edge_weight,
                       b, n, d)


# speedup vs baseline: 1.0026x; 1.0026x over previous
import functools

import jax
import jax.numpy as jnp
from jax import lax
from jax.experimental import pallas as pl
from jax.experimental.pallas import tpu as pltpu
from jax.experimental.pallas import tpu_sc as plsc

NC = 2
NS = 16
LANES = 16

CHUNK = 80
EBLK = 2000
IBLK = 25


def _tc_matmul_split(x, w):
  m, k = x.shape
  _, n = w.shape
  hd = n // NC
  bm = 2000

  def body(x_ref, w_ref, o_ref):
    o_ref[...] = jnp.dot(x_ref[...], w_ref[...],
                         preferred_element_type=jnp.float32)

  return pl.pallas_call(
      body,
      grid=(NC, m // bm),
      in_specs=[
          pl.BlockSpec((bm, k), lambda c, i: (i, 0)),
          pl.BlockSpec((k, hd), lambda c, i: (0, c)),
      ],
      out_specs=pl.BlockSpec((bm, hd), lambda c, i: (c * (m // bm) + i, 0)),
      out_shape=jax.ShapeDtypeStruct((NC * m, hd), jnp.float32),
  )(x, w)


def _sc_aggregate(support2, src, dst, ew, b, n, d):
  hd = d // NC
  e = src.shape[0]
  ept = e // NS
  nblk = ept // EBLK
  nchunk = EBLK // CHUNK
  ninit = n // NS
  dcol = hd // LANES

  mesh = plsc.VectorSubcoreMesh(core_axis_name="c", subcore_axis_name="s")

  @functools.partial(
      pl.kernel,
      out_type=jax.ShapeDtypeStruct((n, d), jnp.float32),
      mesh=mesh,
      compiler_params=pltpu.CompilerParams(needs_layout_passes=False),
      scratch_types=dict(
          acc=pltpu.VMEM_SHARED((n, hd), jnp.float32),
          srcv=pltpu.VMEM((EBLK,), jnp.int32),
          dstv=pltpu.VMEM((EBLK,), jnp.int32),
          ewv=pltpu.VMEM((EBLK,), jnp.float32),
          idxv0=pltpu.VMEM((CHUNK,), jnp.int32),
          idxv1=pltpu.VMEM((CHUNK,), jnp.int32),
          rows0=pltpu.VMEM((CHUNK, hd), jnp.float32),
          rows1=pltpu.VMEM((CHUNK, hd), jnp.float32),
          binit=pltpu.VMEM((IBLK, hd), jnp.float32),
          bvm=pltpu.VMEM((hd,), jnp.float32),
          gsem0=pltpu.SemaphoreType.DMA,
          gsem1=pltpu.SemaphoreType.DMA,
      ),
  )
  def agg(sup_hbm, src_hbm, dst_hbm, ew_hbm, b_hbm, out_hbm,
          acc, srcv, dstv, ewv, idxv0, idxv1, rows0, rows1, binit, bvm,
          gsem0, gsem1):
    cid = lax.axis_index("c")
    sid = lax.axis_index("s")

    pltpu.sync_copy(b_hbm.at[pl.ds(cid * hd, hd)], bvm)

    def binit_body(i, _):
      for r in range(dcol):
        sl = pl.ds(r * LANES, LANES)
        binit[i, sl] = bvm[sl]
      return 0

    lax.fori_loop(0, IBLK, binit_body, 0)

    for j in range(ninit // IBLK):
      pltpu.sync_copy(binit, acc.at[pl.ds(sid * ninit + j * IBLK, IBLK)])

    plsc.subcore_barrier()

    row_off = cid * n

    bufs = ((idxv0, rows0, gsem0), (idxv1, rows1, gsem1))

    def fill_idx(c, idxv):
      cb = c * CHUNK
      for t in range(CHUNK // LANES):
        sl = pl.ds(t * LANES, LANES)
        idxv[sl] = srcv[pl.ds(cb + t * LANES, LANES)] + row_off

    def block_body(bi, _):
      ebase = sid * ept + bi * EBLK
      pltpu.sync_copy(src_hbm.at[pl.ds(ebase, EBLK)], srcv)
      pltpu.sync_copy(dst_hbm.at[pl.ds(ebase, EBLK)], dstv)
      pltpu.sync_copy(ew_hbm.at[pl.ds(ebase, EBLK)], ewv)

      fill_idx(0, idxv0)
      pend = pltpu.async_copy(sup_hbm.at[idxv0], rows0, gsem0)
      for c in range(nchunk):
        cb = c * CHUNK
        idxv, rows, _ = bufs[c % 2]
        pend.wait()
        if c + 1 < nchunk:
          nidxv, nrows, ngsem = bufs[(c + 1) % 2]
          fill_idx(c + 1, nidxv)
          pend = pltpu.async_copy(sup_hbm.at[nidxv], nrows, ngsem)

        def scale_body(t, _, cb=cb, rows=rows):
          wv = ewv[pl.ds(cb + t * LANES, LANES)]
          for li in range(LANES):
            w = wv[li]
            rr = t * LANES + li
            for r in range(dcol):
              sl = pl.ds(r * LANES, LANES)
              rows[rr, sl] = rows[rr, sl] * w
          return 0

        lax.fori_loop(0, CHUNK // LANES, scale_body, 0)
        pltpu.sync_copy(rows, acc.at[dstv.at[pl.ds(cb, CHUNK)]], add=True)
      return 0

    lax.fori_loop(0, nblk, block_body, 0)

    plsc.subcore_barrier()

    wa = 624
    wlast = n - (NS - 1) * wa

    @pl.when(sid < NS - 1)
    def _():
      pltpu.sync_copy(
          acc.at[pl.ds(sid * wa, wa)],
          out_hbm.at[pl.ds(sid * wa, wa), pl.ds(cid * hd, hd)])

    @pl.when(sid == NS - 1)
    def _():
      pltpu.sync_copy(
          acc.at[pl.ds((NS - 1) * wa, wlast)],
          out_hbm.at[pl.ds((NS - 1) * wa, wlast), pl.ds(cid * hd, hd)])

  return agg(support2, src, dst, ew, b)


def kernel(x, edge_index, edge_weight, W, b):
  n, _ = x.shape
  d = W.shape[1]
  support2 = _tc_matmul_split(x, W)
  return _sc_aggregate(support2, edge_index[0], edge_index[1], edge_weight,
                       b, n, d)
